# Initial kernel scaffold; baseline (speedup 1.0000x reference)
import jax
import jax.numpy as jnp
from jax.experimental import pallas as pl


def kernel(x, table):
    raise NotImplementedError("write your pallas kernel here")


# speedup vs baseline: 1.5661x; 1.5661x over previous
import jax
import jax.numpy as jnp
from jax import lax
from jax.experimental import pallas as pl
from jax.experimental.pallas import tpu as pltpu
from jax.experimental.pallas import tpu_sc as plsc

NUM_CLASSES = 1000000
EMBED_DIM = 32
BATCH = 16384
FIELDS = 26

_NC, _NS = 2, 16
_NW = _NC * _NS
_B = BATCH * FIELDS
_BPW = _B // _NW
_CHUNK = 1024
_NCHUNK = _BPW // _CHUNK


def _embed_body(x_hbm, table_hbm, out_hbm, idx_v, rows0, rows1, sem_g0, sem_g1,
                sem_o0, sem_o1):
    wid = lax.axis_index("s") * _NC + lax.axis_index("c")
    base = wid * _BPW
    pltpu.sync_copy(x_hbm.at[pl.ds(base, _BPW)], idx_v)

    rows = (rows0, rows1)
    sem_g = (sem_g0, sem_g1)
    sem_o = (sem_o0, sem_o1)
    out_cps = [None, None]
    for j in range(_NCHUNK):
        b = j & 1
        if out_cps[b] is not None:
            out_cps[b].wait()
        pltpu.async_copy(
            table_hbm.at[idx_v.at[pl.ds(j * _CHUNK, _CHUNK)]],
            rows[b], sem_g[b]).wait()
        out_cps[b] = pltpu.async_copy(
            rows[b], out_hbm.at[pl.ds(base + j * _CHUNK, _CHUNK)], sem_o[b])
    for cp in out_cps:
        if cp is not None:
            cp.wait()


def kernel(x, table):
    mesh = plsc.VectorSubcoreMesh(core_axis_name="c", subcore_axis_name="s",
                                  num_cores=_NC, num_subcores=_NS)
    flat_idx = x.reshape(_B)
    out = pl.kernel(
        _embed_body,
        out_type=jax.ShapeDtypeStruct((_B, EMBED_DIM), jnp.float32),
        mesh=mesh,
        scratch_types=[
            pltpu.VMEM((_BPW,), jnp.int32),
            pltpu.VMEM((_CHUNK, EMBED_DIM), jnp.float32),
            pltpu.VMEM((_CHUNK, EMBED_DIM), jnp.float32),
            pltpu.SemaphoreType.DMA,
            pltpu.SemaphoreType.DMA,
            pltpu.SemaphoreType.DMA,
            pltpu.SemaphoreType.DMA,
        ],
        compiler_params=pltpu.CompilerParams(use_tc_tiling_on_sc=False),
    )(flat_idx, table)
    return out.reshape(BATCH, FIELDS, EMBED_DIM)


# speedup vs baseline: 1.5747x; 1.0055x over previous
import jax
import jax.numpy as jnp
from jax import lax
from jax.experimental import pallas as pl
from jax.experimental.pallas import tpu as pltpu
from jax.experimental.pallas import tpu_sc as plsc

NUM_CLASSES = 1000000
EMBED_DIM = 32
BATCH = 16384
FIELDS = 26

_NC, _NS = 2, 16
_NW = _NC * _NS
_B = BATCH * FIELDS
_BPW = _B // _NW
_CHUNK = 832
_NCHUNK = _BPW // _CHUNK
_NBUF = 4


def _embed_body(x_hbm, table_hbm, out_hbm, idx_v, rows, sems_g, sems_o):
    wid = lax.axis_index("s") * _NC + lax.axis_index("c")
    base = wid * _BPW
    pltpu.sync_copy(x_hbm.at[pl.ds(base, _BPW)], idx_v)

    lag = _NBUF - 1
    g_cps = [None] * _NBUF
    out_cps = [None] * _NBUF
    for j in range(_NCHUNK + lag):
        if j < _NCHUNK:
            b = j % _NBUF
            if out_cps[b] is not None:
                out_cps[b].wait()
            g_cps[b] = pltpu.async_copy(
                table_hbm.at[idx_v.at[pl.ds(j * _CHUNK, _CHUNK)]],
                rows[b], sems_g[b])
        if j >= lag:
            i = j - lag
            b = i % _NBUF
            g_cps[b].wait()
            out_cps[b] = pltpu.async_copy(
                rows[b], out_hbm.at[pl.ds(base + i * _CHUNK, _CHUNK)],
                sems_o[b])
    for cp in out_cps:
        if cp is not None:
            cp.wait()


def kernel(x, table):
    mesh = plsc.VectorSubcoreMesh(core_axis_name="c", subcore_axis_name="s",
                                  num_cores=_NC, num_subcores=_NS)
    flat_idx = x.reshape(_B)
    out = pl.kernel(
        _embed_body,
        out_type=jax.ShapeDtypeStruct((_B, EMBED_DIM), jnp.float32),
        mesh=mesh,
        scratch_types=[
            pltpu.VMEM((_BPW,), jnp.int32),
            [pltpu.VMEM((_CHUNK, EMBED_DIM), jnp.float32)] * _NBUF,
            [pltpu.SemaphoreType.DMA] * _NBUF,
            [pltpu.SemaphoreType.DMA] * _NBUF,
        ],
        compiler_params=pltpu.CompilerParams(use_tc_tiling_on_sc=False),
    )(flat_idx, table)
    return out.reshape(BATCH, FIELDS, EMBED_DIM)


# speedup vs baseline: 1.6739x; 1.0630x over previous
import jax
import jax.numpy as jnp
from jax import lax
from jax.experimental import pallas as pl
from jax.experimental.pallas import tpu as pltpu
from jax.experimental.pallas import tpu_sc as plsc

NUM_CLASSES = 1000000
EMBED_DIM = 32
BATCH = 16384
FIELDS = 26

_NC, _NS = 2, 16
_NW = _NC * _NS
_B = BATCH * FIELDS
_BPW = _B // _NW
_CHUNK = 832
_NCHUNK = _BPW // _CHUNK
_NBUF = 4


def _embed_body(x_hbm, table_hbm, out_hbm, idx_v, rows, sems_g, sems_o):
    wid = lax.axis_index("s") * _NC + lax.axis_index("c")
    base = wid * _BPW
    pltpu.sync_copy(x_hbm.at[pl.ds(base, _BPW)], idx_v)

    lag = _NBUF - 1
    g_cps = [None] * _NBUF
    out_cps = [None] * _NBUF
    for j in range(_NCHUNK + lag):
        if j < _NCHUNK:
            b = j % _NBUF
            if out_cps[b] is not None:
                out_cps[b].wait()
            g_cps[b] = pltpu.async_copy(
                table_hbm.at[idx_v.at[pl.ds(j * _CHUNK, _CHUNK)]],
                rows[b], sems_g[b])
        if j >= lag:
            i = j - lag
            b = i % _NBUF
            g_cps[b].wait()
            out_cps[b] = pltpu.async_copy(
                rows[b], out_hbm.at[pl.ds(base + i * _CHUNK, _CHUNK)],
                sems_o[b])
    for cp in out_cps:
        if cp is not None:
            cp.wait()


def kernel(x, table):
    mesh = plsc.VectorSubcoreMesh(core_axis_name="c", subcore_axis_name="s",
                                  num_cores=_NC, num_subcores=_NS)
    flat_idx = x.T.reshape(_B)
    out = pl.kernel(
        _embed_body,
        out_type=jax.ShapeDtypeStruct((_B, EMBED_DIM), jnp.float32),
        mesh=mesh,
        scratch_types=[
            pltpu.VMEM((_BPW,), jnp.int32),
            [pltpu.VMEM((_CHUNK, EMBED_DIM), jnp.float32)] * _NBUF,
            [pltpu.SemaphoreType.DMA] * _NBUF,
            [pltpu.SemaphoreType.DMA] * _NBUF,
        ],
        compiler_params=pltpu.CompilerParams(use_tc_tiling_on_sc=False),
    )(flat_idx, table)
    return out.reshape(FIELDS, BATCH, EMBED_DIM).transpose(1, 0, 2)
